# baseline (speedup 1.0000x reference)
import functools

import jax
import jax.numpy as jnp
from jax.experimental import pallas as pl
from jax.experimental.pallas import tpu as pltpu

_UPDATE_RATE = 0.5
_AGE_FACTOR = 0.98


def _body(S, B, NS, M,
          hs_ref, mem0_ref,
          wiv_ref, biv_ref, wgur_ref, bgur_ref,
          wrgh_ref, wuh_ref,
          out_ref,
          min_scr, xg_scr, xu_scr, xr_scr):
    f32 = jnp.float32

    hs = hs_ref[...]
    miv = jnp.dot(hs, wiv_ref[...], preferred_element_type=f32) + biv_ref[...]
    vals = miv[:, M:]
    xgur = jnp.dot(vals, wgur_ref[...], preferred_element_type=f32) + bgur_ref[...]
    min_scr[...] = miv[:, :M].reshape(S, B, M)
    xg_scr[...] = xgur[:, :M].reshape(S, B, M)
    xu_scr[...] = xgur[:, M:2 * M].reshape(S, B, M)
    xr_scr[...] = xgur[:, 2 * M:].reshape(S, B, M)

    wrgh = wrgh_ref[...]
    wuh = wuh_ref[...]

    def step(t, carry):
        mem, usage, age = carry
        m_in = min_scr[t]
        xg = xg_scr[t]
        xu = xu_scr[t]
        xr = xr_scr[t]

        sim = jnp.sum(mem * m_in[:, None, :], axis=2)
        scores = usage * 0.2 + age * 0.1 - sim
        w = scores - jnp.max(scores, axis=1, keepdims=True)
        e = jnp.exp(w)
        write_w = e / jnp.sum(e, axis=1, keepdims=True)

        mem2 = mem.reshape(B * NS, M)
        r_pre = jnp.dot(mem2, wrgh[:, :M], preferred_element_type=f32)
        g_pre = jnp.dot(mem2, wrgh[:, M:], preferred_element_type=f32)
        reset = jax.nn.sigmoid(r_pre.reshape(B, NS, M) + xr[:, None, :])
        upd = jax.nn.sigmoid(g_pre.reshape(B, NS, M) + xg[:, None, :])
        rh = (reset * mem).reshape(B * NS, M)
        cand = jnp.tanh(
            jnp.dot(rh, wuh, preferred_element_type=f32).reshape(B, NS, M)
            + xu[:, None, :])

        mask = write_w > 0.01
        wwm = jnp.where(mask, write_w * _UPDATE_RATE, jnp.zeros_like(write_w))
        s = wwm[:, :, None] * upd
        memn = mem + s * (cand - mem)
        usage = (usage + jnp.where(mask, write_w, jnp.zeros_like(write_w))) * 0.99
        nsq = jnp.sum(memn * memn, axis=2, keepdims=True)
        memn = memn * jax.lax.rsqrt(jnp.maximum(nsq, 1e-24))
        age = age * _AGE_FACTOR + 1.0
        return memn, usage, age

    zeros = jnp.zeros((B, NS), dtype=f32)
    mem_final, _, _ = jax.lax.fori_loop(0, S, step, (mem0_ref[...], zeros, zeros))
    out_ref[...] = mem_final


@jax.jit
def kernel(hidden_states, memory0, W_in, b_in, W_val, b_val,
           W_gate, b_gate, W_upd, b_upd, W_reset, b_reset):
    B, S, D = hidden_states.shape
    _, NS, M = memory0.shape

    hs = jnp.transpose(hidden_states, (1, 0, 2)).reshape(S * B, D)
    wiv = jnp.concatenate([W_in.T, W_val.T], axis=1)
    biv = jnp.concatenate([b_in, b_val]).reshape(1, 2 * M)
    wgur = jnp.concatenate(
        [W_gate[:, :M].T, W_upd[:, :M].T, W_reset[:, :M].T], axis=1)
    bgur = jnp.concatenate([b_gate, b_upd, b_reset]).reshape(1, 3 * M)
    wrgh = jnp.concatenate([W_reset[:, M:].T, W_gate[:, M:].T], axis=1)
    wuh = W_upd[:, M:].T

    body = functools.partial(_body, S, B, NS, M)
    out = pl.pallas_call(
        body,
        out_shape=jax.ShapeDtypeStruct((B, NS, M), jnp.float32),
        scratch_shapes=[pltpu.VMEM((S, B, M), jnp.float32)] * 4,
    )(hs, memory0, wiv, biv, wgur, bgur, wrgh, wuh)
    return out


# speedup vs baseline: 1.0012x; 1.0012x over previous
import functools

import jax
import jax.numpy as jnp
from jax.experimental import pallas as pl
from jax.experimental.pallas import tpu as pltpu

_UPDATE_RATE = 0.5
_AGE_FACTOR = 0.98


def _body(S, B, NS, M,
          hs_ref, mem0_ref,
          wiv_ref, biv_ref, wgur_ref, bgur_ref,
          wrgh_ref, wuh_ref,
          out_ref,
          min_scr, xg_scr, xu_scr, xr_scr):
    f32 = jnp.float32

    hs = hs_ref[...]
    wiv = wiv_ref[...]
    wgur = wgur_ref[...]
    m_in_all = jnp.dot(hs, wiv[:, :M], preferred_element_type=f32) + biv_ref[0, :M]
    vals = jnp.dot(hs, wiv[:, M:], preferred_element_type=f32) + biv_ref[0, M:]
    xg_all = jnp.dot(vals, wgur[:, :M], preferred_element_type=f32) + bgur_ref[0, :M]
    xu_all = jnp.dot(vals, wgur[:, M:2 * M], preferred_element_type=f32) + bgur_ref[0, M:2 * M]
    xr_all = jnp.dot(vals, wgur[:, 2 * M:], preferred_element_type=f32) + bgur_ref[0, 2 * M:]
    min_scr[...] = m_in_all.reshape(S, B, M)
    xg_scr[...] = xg_all.reshape(S, B, M)
    xu_scr[...] = xu_all.reshape(S, B, M)
    xr_scr[...] = xr_all.reshape(S, B, M)

    wrgh = wrgh_ref[...]
    wuh = wuh_ref[...]

    def step(t, carry):
        mem, usage, age = carry
        m_in = min_scr[t]
        xg = xg_scr[t]
        xu = xu_scr[t]
        xr = xr_scr[t]

        sim = jnp.sum(mem * m_in[:, None, :], axis=2)
        scores = usage * 0.2 + age * 0.1 - sim
        w = scores - jnp.max(scores, axis=1, keepdims=True)
        e = jnp.exp(w)
        write_w = e / jnp.sum(e, axis=1, keepdims=True)

        mem2 = mem.reshape(B * NS, M)
        r_pre = jnp.dot(mem2, wrgh[:, :M], preferred_element_type=f32)
        g_pre = jnp.dot(mem2, wrgh[:, M:], preferred_element_type=f32)
        reset = jax.nn.sigmoid(r_pre.reshape(B, NS, M) + xr[:, None, :])
        upd = jax.nn.sigmoid(g_pre.reshape(B, NS, M) + xg[:, None, :])
        rh = (reset * mem).reshape(B * NS, M)
        cand = jnp.tanh(
            jnp.dot(rh, wuh, preferred_element_type=f32).reshape(B, NS, M)
            + xu[:, None, :])

        mask = write_w > 0.01
        wwm = jnp.where(mask, write_w * _UPDATE_RATE, jnp.zeros_like(write_w))
        s = wwm[:, :, None] * upd
        memn = mem + s * (cand - mem)
        usage = (usage + jnp.where(mask, write_w, jnp.zeros_like(write_w))) * 0.99
        nsq = jnp.sum(memn * memn, axis=2, keepdims=True)
        memn = memn * jax.lax.rsqrt(jnp.maximum(nsq, 1e-24))
        age = age * _AGE_FACTOR + 1.0
        return memn, usage, age

    zeros = jnp.zeros((B, NS), dtype=f32)
    mem_final, _, _ = jax.lax.fori_loop(0, S, step, (mem0_ref[...], zeros, zeros))
    out_ref[...] = mem_final


@jax.jit
def kernel(hidden_states, memory0, W_in, b_in, W_val, b_val,
           W_gate, b_gate, W_upd, b_upd, W_reset, b_reset):
    B, S, D = hidden_states.shape
    _, NS, M = memory0.shape

    hs = jnp.transpose(hidden_states, (1, 0, 2)).reshape(S * B, D)
    wiv = jnp.concatenate([W_in.T, W_val.T], axis=1)
    biv = jnp.concatenate([b_in, b_val]).reshape(1, 2 * M)
    wgur = jnp.concatenate(
        [W_gate[:, :M].T, W_upd[:, :M].T, W_reset[:, :M].T], axis=1)
    bgur = jnp.concatenate([b_gate, b_upd, b_reset]).reshape(1, 3 * M)
    wrgh = jnp.concatenate([W_reset[:, M:].T, W_gate[:, M:].T], axis=1)
    wuh = W_upd[:, M:].T

    body = functools.partial(_body, S, B, NS, M)
    out = pl.pallas_call(
        body,
        out_shape=jax.ShapeDtypeStruct((B, NS, M), jnp.float32),
        scratch_shapes=[pltpu.VMEM((S, B, M), jnp.float32)] * 4,
    )(hs, memory0, wiv, biv, wgur, bgur, wrgh, wuh)
    return out


# speedup vs baseline: 1.0917x; 1.0904x over previous
import functools

import jax
import jax.numpy as jnp
from jax.experimental import pallas as pl
from jax.experimental.pallas import tpu as pltpu

_UPDATE_RATE = 0.5
_AGE_FACTOR = 0.98


def _body(S, B, NS, M,
          hs_ref, mem0_ref,
          win_ref, wval_ref,
          wgx_ref, wgh_ref, wux_ref, wuh_ref, wrx_ref, wrh_ref,
          bin_ref, bval_ref, bg_ref, bu_ref, br_ref,
          out_ref,
          min_scr, xg_scr, xu_scr, xr_scr):
    f32 = jnp.float32

    hs = hs_ref[...]
    m_in_all = jnp.dot(hs, win_ref[...], preferred_element_type=f32) + bin_ref[...]
    vals = jnp.dot(hs, wval_ref[...], preferred_element_type=f32) + bval_ref[...]
    xg_all = jnp.dot(vals, wgx_ref[...], preferred_element_type=f32) + bg_ref[...]
    xu_all = jnp.dot(vals, wux_ref[...], preferred_element_type=f32) + bu_ref[...]
    xr_all = jnp.dot(vals, wrx_ref[...], preferred_element_type=f32) + br_ref[...]
    min_scr[...] = m_in_all.reshape(S, B, M)
    xg_scr[...] = xg_all.reshape(S, B, M)
    xu_scr[...] = xu_all.reshape(S, B, M)
    xr_scr[...] = xr_all.reshape(S, B, M)

    wgh = wgh_ref[...]
    wuh = wuh_ref[...]
    wrh = wrh_ref[...]

    def step(t, carry):
        mem, usage, age = carry
        m_in = min_scr[t]
        xg = xg_scr[t]
        xu = xu_scr[t]
        xr = xr_scr[t]

        sim = jnp.sum(mem * m_in[:, None, :], axis=2)
        scores = usage * 0.2 + age * 0.1 - sim
        w = scores - jnp.max(scores, axis=1, keepdims=True)
        e = jnp.exp(w)
        write_w = e / jnp.sum(e, axis=1, keepdims=True)

        mem2 = mem.reshape(B * NS, M)
        reset = jax.nn.sigmoid(
            jnp.dot(mem2, wrh, preferred_element_type=f32).reshape(B, NS, M)
            + xr[:, None, :])
        upd = jax.nn.sigmoid(
            jnp.dot(mem2, wgh, preferred_element_type=f32).reshape(B, NS, M)
            + xg[:, None, :])
        rh = (reset * mem).reshape(B * NS, M)
        cand = jnp.tanh(
            jnp.dot(rh, wuh, preferred_element_type=f32).reshape(B, NS, M)
            + xu[:, None, :])
        new_h = (1.0 - upd) * mem + upd * cand

        ww = write_w[:, :, None] * _UPDATE_RATE
        updated = mem * (1.0 - ww) + new_h * ww
        mask = write_w > 0.01
        memn = jnp.where(mask[:, :, None], updated, mem)
        usage = usage + jnp.where(mask, write_w, jnp.zeros_like(write_w))
        norm = jnp.sqrt(jnp.sum(memn * memn, axis=2, keepdims=True))
        memn = memn / jnp.maximum(norm, 1e-12)
        age = age * _AGE_FACTOR + 1.0
        usage = usage * 0.99
        return memn, usage, age

    zeros = jnp.zeros((B, NS), dtype=f32)
    mem_final, _, _ = jax.lax.fori_loop(0, S, step, (mem0_ref[...], zeros, zeros))
    out_ref[...] = mem_final


@jax.jit
def kernel(hidden_states, memory0, W_in, b_in, W_val, b_val,
           W_gate, b_gate, W_upd, b_upd, W_reset, b_reset):
    B, S, D = hidden_states.shape
    _, NS, M = memory0.shape

    hs = jnp.transpose(hidden_states, (1, 0, 2)).reshape(S * B, D)
    win_t = W_in.T
    wval_t = W_val.T
    wgx, wgh = W_gate[:, :M].T, W_gate[:, M:].T
    wux, wuh = W_upd[:, :M].T, W_upd[:, M:].T
    wrx, wrh = W_reset[:, :M].T, W_reset[:, M:].T

    body = functools.partial(_body, S, B, NS, M)
    out = pl.pallas_call(
        body,
        out_shape=jax.ShapeDtypeStruct((B, NS, M), jnp.float32),
        scratch_shapes=[pltpu.VMEM((S, B, M), jnp.float32)] * 4,
    )(hs, memory0,
      win_t, wval_t, wgx, wgh, wux, wuh, wrx, wrh,
      b_in.reshape(1, M), b_val.reshape(1, M), b_gate.reshape(1, M),
      b_upd.reshape(1, M), b_reset.reshape(1, M))
    return out
